# baseline (speedup 1.0000x reference)
import functools

import jax
import jax.numpy as jnp
from jax import lax
from jax.experimental import pallas as pl
from jax.experimental.pallas import tpu as pltpu
from jax.experimental.pallas import tpu_sc as plsc

_SLEN = 20
_VOCAB = 100000
_PAD = 32
_LANES = 16


def _log16(x):
    bits = lax.bitcast_convert_type(x, jnp.int32)
    e = (bits >> 23) - 127
    m = lax.bitcast_convert_type((bits & 0x007FFFFF) | 0x3F800000, jnp.float32)
    big = m > 1.41421356237
    m = jnp.where(big, m * 0.5, m)
    ef = jnp.where(big, e + 1, e).astype(jnp.float32)
    t = m - 1.0
    z = t * t
    p = jnp.full((_LANES,), 7.0376836292e-2, jnp.float32)
    for c in (-1.1514610310e-1, 1.1676998740e-1, -1.2420140846e-1,
              1.4249322787e-1, -1.6668057665e-1, 2.0000714765e-1,
              -2.4999993993e-1, 3.3333331174e-1):
        p = p * t + c
    y = t * z * p - 0.5 * z
    return t + y + ef * 0.69314718056


@functools.partial(
    pl.kernel,
    out_type=jax.ShapeDtypeStruct((_LANES,), jnp.float32),
    mesh=plsc.VectorSubcoreMesh(core_axis_name="c", subcore_axis_name="s"),
    scratch_types=[
        pltpu.VMEM((_PAD,), jnp.int32),
        pltpu.VMEM((_PAD,), jnp.int32),
        pltpu.VMEM((_PAD,), jnp.float32),
        pltpu.VMEM((_PAD,), jnp.float32),
        pltpu.VMEM((_LANES,), jnp.float32),
        pltpu.SemaphoreType.DMA,
    ],
)
def _pll_score(probs_hbm, ids_hbm, out_hbm, ids_v, idx_v, vals_v, red_v,
               out_v, sem):
    cid = lax.axis_index("c")
    sid = lax.axis_index("s")

    @pl.when(jnp.logical_and(cid == 0, sid == 0))
    def _():
        pltpu.sync_copy(ids_hbm, ids_v)
        for j in range(_PAD // _LANES):
            ids16 = ids_v[pl.ds(j * _LANES, _LANES)]
            pos = lax.iota(jnp.int32, _LANES) + (j * _LANES)
            valid = jnp.logical_and(pos < _SLEN, ids16 >= 0)
            base = (pos * (_SLEN + 1) + _SLEN) * _VOCAB
            flat = jnp.where(valid, base + jnp.where(valid, ids16, 0), 0)
            idx_v[pl.ds(j * _LANES, _LANES)] = flat
        pltpu.async_copy(probs_hbm.at[idx_v], vals_v, sem).wait()
        acc = jnp.zeros((_LANES,), jnp.float32)
        cnt = jnp.zeros((_LANES,), jnp.float32)
        for j in range(_PAD // _LANES):
            x = vals_v[pl.ds(j * _LANES, _LANES)]
            ids16 = ids_v[pl.ds(j * _LANES, _LANES)]
            pos = lax.iota(jnp.int32, _LANES) + (j * _LANES)
            valid = jnp.logical_and(pos < _SLEN, ids16 >= 0)
            acc = acc + jnp.where(valid, _log16(x), 0.0)
            cnt = cnt + jnp.where(valid, 1.0, 0.0)
        total = acc[0]
        count = cnt[0]
        for i in range(1, _LANES):
            total = total + acc[i]
            count = count + cnt[i]
        out_v[...] = (jnp.full((_LANES,), total, jnp.float32) /
                      jnp.full((_LANES,), count, jnp.float32))
        pltpu.sync_copy(out_v, out_hbm)


def kernel(probs, origids):
    ids = jnp.full((_PAD,), -1, jnp.int32).at[:_SLEN].set(
        origids.astype(jnp.int32))
    out = _pll_score(probs.reshape(-1), ids)
    return out[0]


# speedup vs baseline: 14.5742x; 14.5742x over previous
import jax
import jax.numpy as jnp
from jax import lax
from jax.experimental import pallas as pl
from jax.experimental.pallas import tpu as pltpu

_SLEN = 20
_VOCAB = 100000
_B = 512


def _body(ids_ref, block_ref, out_ref, acc_ref, cnt_ref):
    i = pl.program_id(0)

    @pl.when(i == 0)
    def _():
        acc_ref[0] = 0.0
        cnt_ref[0] = 0.0

    idv = ids_ref[i]
    valid = idv >= 0
    off = lax.rem(jnp.maximum(idv, 0), _B)
    x = block_ref[...]
    row = lax.broadcasted_iota(jnp.int32, (1, _SLEN, _B), 1) == i
    lane = lax.broadcasted_iota(jnp.int32, (1, _SLEN, _B), 2) == off
    sel = jnp.logical_and(row, lane)
    lg = jnp.log(jnp.where(sel, x, 1.0))
    contrib = jnp.sum(jnp.where(sel, lg, 0.0))
    acc_ref[0] = acc_ref[0] + jnp.where(valid, contrib, 0.0)
    cnt_ref[0] = cnt_ref[0] + jnp.where(valid, 1.0, 0.0)

    @pl.when(i == _SLEN - 1)
    def _():
        out_ref[0] = acc_ref[0] / cnt_ref[0]


def kernel(probs, origids):
    ids = origids.astype(jnp.int32)
    grid_spec = pltpu.PrefetchScalarGridSpec(
        num_scalar_prefetch=1,
        grid=(_SLEN,),
        in_specs=[
            pl.BlockSpec(
                (1, _SLEN, _B),
                lambda i, ids_ref: (i + 1, 0, jnp.maximum(ids_ref[i], 0) // _B),
            ),
        ],
        out_specs=pl.BlockSpec(memory_space=pltpu.SMEM),
        scratch_shapes=[
            pltpu.SMEM((1,), jnp.float32),
            pltpu.SMEM((1,), jnp.float32),
        ],
    )
    out = pl.pallas_call(
        _body,
        grid_spec=grid_spec,
        out_shape=jax.ShapeDtypeStruct((1,), jnp.float32),
    )(ids, probs)
    return out[0]


# speedup vs baseline: 188.9596x; 12.9653x over previous
import jax
import jax.numpy as jnp
from jax import lax
from jax.experimental import pallas as pl
from jax.experimental.pallas import tpu as pltpu

_SLEN = 20
_VOCAB = 100000
_B = 512


def _body(ids_ref, block_ref, out_ref, acc_ref, cnt_ref):
    i = pl.program_id(0)

    @pl.when(i == 0)
    def _():
        acc_ref[0] = 0.0
        cnt_ref[0] = 0.0

    idv = ids_ref[i]
    valid = idv >= 0
    off = lax.rem(jnp.maximum(idv, 0), _B)
    x = block_ref[...]
    row = lax.broadcasted_iota(jnp.int32, (1, _SLEN + 1, _B), 1) == i + 1
    lane = lax.broadcasted_iota(jnp.int32, (1, _SLEN + 1, _B), 2) == off
    sel = jnp.logical_and(row, lane)
    lg = jnp.log(jnp.where(sel, x, 1.0))
    contrib = jnp.sum(jnp.where(sel, lg, 0.0))
    acc_ref[0] = acc_ref[0] + jnp.where(valid, contrib, 0.0)
    cnt_ref[0] = cnt_ref[0] + jnp.where(valid, 1.0, 0.0)

    @pl.when(i == _SLEN - 1)
    def _():
        out_ref[0] = acc_ref[0] / cnt_ref[0]


def kernel(probs, origids):
    ids = origids.astype(jnp.int32)
    probs_t = jnp.transpose(probs, (1, 0, 2))
    grid_spec = pltpu.PrefetchScalarGridSpec(
        num_scalar_prefetch=1,
        grid=(_SLEN,),
        in_specs=[
            pl.BlockSpec(
                (1, _SLEN + 1, _B),
                lambda i, ids_ref: (i, 0, jnp.maximum(ids_ref[i], 0) // _B),
            ),
        ],
        out_specs=pl.BlockSpec(memory_space=pltpu.SMEM),
        scratch_shapes=[
            pltpu.SMEM((1,), jnp.float32),
            pltpu.SMEM((1,), jnp.float32),
        ],
    )
    out = pl.pallas_call(
        _body,
        grid_spec=grid_spec,
        out_shape=jax.ShapeDtypeStruct((1,), jnp.float32),
    )(ids, probs_t)
    return out[0]


# speedup vs baseline: 205.6334x; 1.0882x over previous
import jax
import jax.numpy as jnp
from jax import lax
from jax.experimental import pallas as pl
from jax.experimental.pallas import tpu as pltpu

_SLEN = 20
_VOCAB = 100000
_B = 128


def _body(ids_ref, block_ref, out_ref, acc_ref, cnt_ref):
    i = pl.program_id(0)

    @pl.when(i == 0)
    def _():
        acc_ref[0] = 0.0
        cnt_ref[0] = 0.0

    idv = ids_ref[i]
    valid = idv >= 0
    off = lax.rem(jnp.maximum(idv, 0), _B)
    x = block_ref[...]
    row = lax.broadcasted_iota(jnp.int32, (1, _SLEN + 1, _B), 1) == i + 1
    lane = lax.broadcasted_iota(jnp.int32, (1, _SLEN + 1, _B), 2) == off
    sel = jnp.logical_and(row, lane)
    lg = jnp.log(jnp.where(sel, x, 1.0))
    contrib = jnp.sum(jnp.where(sel, lg, 0.0))
    acc_ref[0] = acc_ref[0] + jnp.where(valid, contrib, 0.0)
    cnt_ref[0] = cnt_ref[0] + jnp.where(valid, 1.0, 0.0)

    @pl.when(i == _SLEN - 1)
    def _():
        out_ref[0] = acc_ref[0] / cnt_ref[0]


def kernel(probs, origids):
    ids = origids.astype(jnp.int32)
    probs_t = jnp.transpose(probs, (1, 0, 2))
    grid_spec = pltpu.PrefetchScalarGridSpec(
        num_scalar_prefetch=1,
        grid=(_SLEN,),
        in_specs=[
            pl.BlockSpec(
                (1, _SLEN + 1, _B),
                lambda i, ids_ref: (i, 0, jnp.maximum(ids_ref[i], 0) // _B),
            ),
        ],
        out_specs=pl.BlockSpec(memory_space=pltpu.SMEM),
        scratch_shapes=[
            pltpu.SMEM((1,), jnp.float32),
            pltpu.SMEM((1,), jnp.float32),
        ],
    )
    out = pl.pallas_call(
        _body,
        grid_spec=grid_spec,
        out_shape=jax.ShapeDtypeStruct((1,), jnp.float32),
    )(ids, probs_t)
    return out[0]


# speedup vs baseline: 884.1217x; 4.2995x over previous
import jax
import jax.numpy as jnp
from jax import lax
from jax.experimental import pallas as pl
from jax.experimental.pallas import tpu as pltpu

_SLEN = 20
_VOCAB = 100000
_B = 128


def _body(ids_ref, *refs):
    blocks = refs[:_SLEN]
    out_ref = refs[_SLEN]
    acc = jnp.zeros((1, _SLEN + 1, _B), jnp.float32)
    cnt = jnp.float32(0.0)
    rows = lax.broadcasted_iota(jnp.int32, (1, _SLEN + 1, _B), 1)
    lanes = lax.broadcasted_iota(jnp.int32, (1, _SLEN + 1, _B), 2)
    for k in range(_SLEN):
        idv = ids_ref[k]
        valid = idv >= 0
        off = lax.rem(jnp.maximum(idv, 0), _B)
        sel = jnp.logical_and(
            jnp.logical_and(rows == k + 1, lanes == off), valid)
        x = blocks[k][...]
        acc = acc + jnp.where(sel, jnp.log(jnp.where(sel, x, 1.0)), 0.0)
        cnt = cnt + jnp.where(valid, 1.0, 0.0)
    out_ref[0] = jnp.sum(acc) / cnt


def kernel(probs, origids):
    ids = origids.astype(jnp.int32)
    probs_t = jnp.transpose(probs, (1, 0, 2))

    def _mk_spec(k):
        return pl.BlockSpec(
            (1, _SLEN + 1, _B),
            lambda i, ids_ref, k=k: (k, 0, jnp.maximum(ids_ref[k], 0) // _B),
        )

    grid_spec = pltpu.PrefetchScalarGridSpec(
        num_scalar_prefetch=1,
        grid=(1,),
        in_specs=[_mk_spec(k) for k in range(_SLEN)],
        out_specs=pl.BlockSpec(memory_space=pltpu.SMEM),
    )
    out = pl.pallas_call(
        _body,
        grid_spec=grid_spec,
        out_shape=jax.ShapeDtypeStruct((1,), jnp.float32),
    )(ids, *([probs_t] * _SLEN))
    return out[0]


# speedup vs baseline: 901.5612x; 1.0197x over previous
import jax
import jax.numpy as jnp
from jax import lax
from jax.experimental import pallas as pl
from jax.experimental.pallas import tpu as pltpu

_SLEN = 20
_VOCAB = 100000
_B = 128
_ROWS = 8
_GRP = 5


def _body(ids_ref, *refs):
    blocks = refs[:_SLEN]
    out_ref = refs[_SLEN]
    rows = lax.broadcasted_iota(jnp.int32, (1, _ROWS, _B), 1)
    lanes = lax.broadcasted_iota(jnp.int32, (1, _ROWS, _B), 2)
    acc = jnp.zeros((1, _ROWS, _B), jnp.float32)
    cnt = jnp.float32(0.0)
    for g in range(_SLEN // _GRP):
        v = jnp.ones((1, _ROWS, _B), jnp.float32)
        for k in range(g * _GRP, (g + 1) * _GRP):
            idv = ids_ref[k]
            valid = idv >= 0
            off = lax.rem(jnp.maximum(idv, 0), _B)
            sel = jnp.logical_and(
                jnp.logical_and(rows == (k + 1) % _ROWS, lanes == off), valid)
            v = v * jnp.where(sel, blocks[k][...], 1.0)
            cnt = cnt + jnp.where(valid, 1.0, 0.0)
        acc = acc + jnp.log(v)
    out_ref[0] = jnp.sum(acc) / cnt


def kernel(probs, origids):
    ids = origids.astype(jnp.int32)
    probs_t = jnp.transpose(probs, (1, 0, 2))

    def _mk_spec(k):
        return pl.BlockSpec(
            (1, _ROWS, _B),
            lambda i, ids_ref, k=k: (
                k, (k + 1) // _ROWS, jnp.maximum(ids_ref[k], 0) // _B),
        )

    grid_spec = pltpu.PrefetchScalarGridSpec(
        num_scalar_prefetch=1,
        grid=(1,),
        in_specs=[_mk_spec(k) for k in range(_SLEN)],
        out_specs=pl.BlockSpec(memory_space=pltpu.SMEM),
    )
    out = pl.pallas_call(
        _body,
        grid_spec=grid_spec,
        out_shape=jax.ShapeDtypeStruct((1,), jnp.float32),
    )(ids, *([probs_t] * _SLEN))
    return out[0]
